# baseline (speedup 1.0000x reference)
import functools

import jax
import jax.numpy as jnp
import numpy as np
from jax import lax
from jax.experimental import pallas as pl
from jax.experimental.pallas import tpu as pltpu
from jax.experimental.pallas import tpu_sc as plsc

_N_EXPERTS = 8
_N_ACTIVE = 5
_TEMP = float(np.e)
_N_TOK = 2048

_INFO = plsc.get_sparse_core_info()
_NC, _NS, _L = _INFO.num_cores, _INFO.num_subcores, _INFO.num_lanes
_NW = _NC * _NS
_TPW = _N_TOK // _NW


def _scores_body(x_ref, gw_ref, gb_ref, st_ref):
    st_ref[...] = (lax.dot_general(
        gw_ref[...], x_ref[...], (((0,), (1,)), ((), ())),
        preferred_element_type=jnp.float32)
        + gb_ref[...].reshape(_N_EXPERTS, 1)) / _TEMP


def _cast_body(x_ref, xb_ref):
    xb_ref[...] = x_ref[...].astype(jnp.bfloat16)


_gate_mesh = plsc.VectorSubcoreMesh(core_axis_name="c", subcore_axis_name="s")


@functools.partial(
    pl.kernel, mesh=_gate_mesh,
    out_type=jax.ShapeDtypeStruct((_N_EXPERTS * _N_TOK,), jnp.float32),
    scratch_types=[
        pltpu.VMEM((_N_EXPERTS, _TPW), jnp.float32),
        pltpu.VMEM((_N_EXPERTS, _TPW), jnp.float32),
        pltpu.SemaphoreType.DMA,
    ],
)
def _gate_sc(st_hbm, w_hbm, sc_v, w_v, sem):
    wid = lax.axis_index("s") * _NC + lax.axis_index("c")
    base = wid * _TPW
    copies = [pltpu.make_async_copy(
        st_hbm.at[pl.ds(i * _N_TOK + base, _TPW)], sc_v.at[i], sem)
        for i in range(_N_EXPERTS)]
    for cp in copies:
        cp.start()
    for cp in copies:
        cp.wait()
    for c in range(_TPW // _L):
        sl = pl.ds(c * _L, _L)
        p = [sc_v[i, sl] for i in range(_N_EXPERTS)]
        m = p[0]
        for i in range(1, _N_EXPERTS):
            m = jnp.maximum(m, p[i])
        ex = [jnp.exp(v - m) for v in p]
        tot = ex[0]
        for i in range(1, _N_EXPERTS):
            tot = tot + ex[i]
        pr = [v / tot for v in ex]
        w_cols = []
        wsum = None
        for i in range(_N_EXPERTS):
            beats = None
            for j in range(_N_EXPERTS):
                if j == i:
                    continue
                if j < i:
                    b = jnp.where(pr[j] >= pr[i], 1.0, 0.0)
                else:
                    b = jnp.where(pr[j] > pr[i], 1.0, 0.0)
                beats = b if beats is None else beats + b
            wi = jnp.where(beats < float(_N_ACTIVE), pr[i], 0.0)
            w_cols.append(wi)
            wsum = wi if wsum is None else wsum + wi
        inv = 1.0 / (wsum + 1e-8)
        for i in range(_N_EXPERTS):
            w_v[i, sl] = w_cols[i] * inv
    out_copies = [pltpu.make_async_copy(
        w_v.at[i], w_hbm.at[pl.ds(i * _N_TOK + base, _TPW)], sem)
        for i in range(_N_EXPERTS)]
    for cp in out_copies:
        cp.start()
    for cp in out_copies:
        cp.wait()


def _experts_body(xb_ref, wts_ref, w1_ref, b1_ref, w2_ref, b2_ref,
                  w3_ref, b3_ref, out_ref, wtsN_ref):
    e = pl.program_id(0)

    @pl.when(e == 0)
    def _tr():
        wtsN_ref[...] = jnp.transpose(wts_ref[...])

    w1b = w1_ref[0].astype(jnp.bfloat16)
    w2b = w2_ref[0].astype(jnp.bfloat16)
    w3b = w3_ref[0].astype(jnp.bfloat16)

    h1 = jnp.maximum(
        jnp.dot(xb_ref[...], w1b, preferred_element_type=jnp.float32)
        + b1_ref[0], 0.0).astype(jnp.bfloat16)
    h2 = jnp.maximum(
        jnp.dot(h1, w2b, preferred_element_type=jnp.float32)
        + b2_ref[0], 0.0).astype(jnp.bfloat16)
    o = jnp.dot(h2, w3b, preferred_element_type=jnp.float32) + b3_ref[0]

    onehot = (jax.lax.broadcasted_iota(jnp.int32, (_N_TOK, _N_EXPERTS), 1)
              == e).astype(jnp.float32)
    w_col = jnp.sum(wtsN_ref[...] * onehot, axis=-1, keepdims=True)
    contrib = w_col * o

    @pl.when(e == 0)
    def _init():
        out_ref[...] = contrib

    @pl.when(e != 0)
    def _acc():
        out_ref[...] += contrib


def kernel(x, gate_W, gate_b, W1, b1, W2, b2, W3, b3):
    n, d = x.shape
    st = pl.pallas_call(
        _scores_body,
        out_shape=jax.ShapeDtypeStruct((_N_EXPERTS, n), jnp.float32),
    )(x, gate_W, gate_b.reshape(1, -1))

    xb = pl.pallas_call(
        _cast_body,
        out_shape=jax.ShapeDtypeStruct((n, d), jnp.bfloat16),
    )(x)

    wts = _gate_sc(st.reshape(-1)).reshape(_N_EXPERTS, _N_TOK)

    return pl.pallas_call(
        _experts_body,
        grid=(_N_EXPERTS,),
        in_specs=[
            pl.BlockSpec((_N_TOK, d), lambda e: (0, 0)),
            pl.BlockSpec((_N_EXPERTS, _N_TOK), lambda e: (0, 0)),
            pl.BlockSpec((1, d, W1.shape[2]), lambda e: (e, 0, 0)),
            pl.BlockSpec((1, 1, b1.shape[1]), lambda e: (e, 0, 0)),
            pl.BlockSpec((1, W2.shape[1], W2.shape[2]), lambda e: (e, 0, 0)),
            pl.BlockSpec((1, 1, b2.shape[1]), lambda e: (e, 0, 0)),
            pl.BlockSpec((1, W3.shape[1], W3.shape[2]), lambda e: (e, 0, 0)),
            pl.BlockSpec((1, 1, b3.shape[1]), lambda e: (e, 0, 0)),
        ],
        out_specs=pl.BlockSpec((_N_TOK, W3.shape[2]), lambda e: (0, 0)),
        out_shape=jax.ShapeDtypeStruct((n, W3.shape[2]), jnp.float32),
        scratch_shapes=[pltpu.VMEM((_N_TOK, _N_EXPERTS), jnp.float32)],
        compiler_params=pltpu.CompilerParams(
            dimension_semantics=("arbitrary",),
            vmem_limit_bytes=100 * 1024 * 1024,
        ),
    )(xb, wts, W1, b1[:, None, :], W2, b2[:, None, :], W3, b3[:, None, :])


# speedup vs baseline: 1.0465x; 1.0465x over previous
import functools

import jax
import jax.numpy as jnp
import numpy as np
from jax import lax
from jax.experimental import pallas as pl
from jax.experimental.pallas import tpu as pltpu
from jax.experimental.pallas import tpu_sc as plsc

_N_EXPERTS = 8
_N_ACTIVE = 5
_TEMP = float(np.e)
_N_TOK = 2048

_INFO = plsc.get_sparse_core_info()
_NC, _NS, _L = _INFO.num_cores, _INFO.num_subcores, _INFO.num_lanes
_NW = _NC * _NS
_TPW = _N_TOK // _NW


def _scores_body(x_ref, gw_ref, gb_ref, st_ref, xb_ref):
    xb_ref[...] = x_ref[...].astype(jnp.bfloat16)
    st_ref[...] = (lax.dot_general(
        gw_ref[...], x_ref[...], (((0,), (1,)), ((), ())),
        preferred_element_type=jnp.float32)
        + gb_ref[...].reshape(_N_EXPERTS, 1)) / _TEMP


_gate_mesh = plsc.VectorSubcoreMesh(core_axis_name="c", subcore_axis_name="s")


@functools.partial(
    pl.kernel, mesh=_gate_mesh,
    out_type=jax.ShapeDtypeStruct((_N_EXPERTS, _N_TOK), jnp.float32),
    scratch_types=[
        pltpu.VMEM((_N_EXPERTS, _TPW), jnp.float32),
        pltpu.VMEM((_N_EXPERTS, _TPW), jnp.float32),
        pltpu.SemaphoreType.DMA,
    ],
)
def _gate_sc(st_hbm, w_hbm, sc_v, w_v, sem):
    wid = lax.axis_index("s") * _NC + lax.axis_index("c")
    base = wid * _TPW
    copies = [pltpu.make_async_copy(
        st_hbm.at[i, pl.ds(base, _TPW)], sc_v.at[i], sem)
        for i in range(_N_EXPERTS)]
    for cp in copies:
        cp.start()
    for cp in copies:
        cp.wait()
    for c in range(_TPW // _L):
        sl = pl.ds(c * _L, _L)
        p = [sc_v[i, sl] for i in range(_N_EXPERTS)]
        m = p[0]
        for i in range(1, _N_EXPERTS):
            m = jnp.maximum(m, p[i])
        ex = [jnp.exp(v - m) for v in p]
        tot = ex[0]
        for i in range(1, _N_EXPERTS):
            tot = tot + ex[i]
        pr = [v / tot for v in ex]
        w_cols = []
        wsum = None
        for i in range(_N_EXPERTS):
            beats = None
            for j in range(_N_EXPERTS):
                if j == i:
                    continue
                if j < i:
                    b = jnp.where(pr[j] >= pr[i], 1.0, 0.0)
                else:
                    b = jnp.where(pr[j] > pr[i], 1.0, 0.0)
                beats = b if beats is None else beats + b
            wi = jnp.where(beats < float(_N_ACTIVE), pr[i], 0.0)
            w_cols.append(wi)
            wsum = wi if wsum is None else wsum + wi
        inv = 1.0 / (wsum + 1e-8)
        for i in range(_N_EXPERTS):
            w_v[i, sl] = w_cols[i] * inv
    out_copies = [pltpu.make_async_copy(
        w_v.at[i], w_hbm.at[i, pl.ds(base, _TPW)], sem)
        for i in range(_N_EXPERTS)]
    for cp in out_copies:
        cp.start()
    for cp in out_copies:
        cp.wait()


def _experts_body(xb_ref, wts_ref, w1_ref, b1_ref, w2_ref, b2_ref,
                  w3_ref, b3_ref, out_ref, wtsN_ref):
    e = pl.program_id(0)

    @pl.when(e == 0)
    def _tr():
        wtsN_ref[...] = jnp.transpose(wts_ref[...])

    w1b = w1_ref[0].astype(jnp.bfloat16)
    w2b = w2_ref[0].astype(jnp.bfloat16)
    w3b = w3_ref[0].astype(jnp.bfloat16)

    h1 = jnp.maximum(
        jnp.dot(xb_ref[...], w1b, preferred_element_type=jnp.float32)
        + b1_ref[0], 0.0).astype(jnp.bfloat16)
    h2 = jnp.maximum(
        jnp.dot(h1, w2b, preferred_element_type=jnp.float32)
        + b2_ref[0], 0.0).astype(jnp.bfloat16)
    o = jnp.dot(h2, w3b, preferred_element_type=jnp.float32) + b3_ref[0]

    onehot = (jax.lax.broadcasted_iota(jnp.int32, (_N_TOK, _N_EXPERTS), 1)
              == e).astype(jnp.float32)
    w_col = jnp.sum(wtsN_ref[...] * onehot, axis=-1, keepdims=True)
    contrib = w_col * o

    @pl.when(e == 0)
    def _init():
        out_ref[...] = contrib

    @pl.when(e != 0)
    def _acc():
        out_ref[...] += contrib


def kernel(x, gate_W, gate_b, W1, b1, W2, b2, W3, b3):
    n, d = x.shape
    st, xb = pl.pallas_call(
        _scores_body,
        out_shape=[
            jax.ShapeDtypeStruct((_N_EXPERTS, n), jnp.float32),
            jax.ShapeDtypeStruct((n, d), jnp.bfloat16),
        ],
    )(x, gate_W, gate_b.reshape(1, -1))

    wts = _gate_sc(st)

    return pl.pallas_call(
        _experts_body,
        grid=(_N_EXPERTS,),
        in_specs=[
            pl.BlockSpec((_N_TOK, d), lambda e: (0, 0)),
            pl.BlockSpec((_N_EXPERTS, _N_TOK), lambda e: (0, 0)),
            pl.BlockSpec((1, d, W1.shape[2]), lambda e: (e, 0, 0)),
            pl.BlockSpec((1, 1, b1.shape[1]), lambda e: (e, 0, 0)),
            pl.BlockSpec((1, W2.shape[1], W2.shape[2]), lambda e: (e, 0, 0)),
            pl.BlockSpec((1, 1, b2.shape[1]), lambda e: (e, 0, 0)),
            pl.BlockSpec((1, W3.shape[1], W3.shape[2]), lambda e: (e, 0, 0)),
            pl.BlockSpec((1, 1, b3.shape[1]), lambda e: (e, 0, 0)),
        ],
        out_specs=pl.BlockSpec((_N_TOK, W3.shape[2]), lambda e: (0, 0)),
        out_shape=jax.ShapeDtypeStruct((n, W3.shape[2]), jnp.float32),
        scratch_shapes=[pltpu.VMEM((_N_TOK, _N_EXPERTS), jnp.float32)],
        compiler_params=pltpu.CompilerParams(
            dimension_semantics=("arbitrary",),
            vmem_limit_bytes=100 * 1024 * 1024,
        ),
    )(xb, wts, W1, b1[:, None, :], W2, b2[:, None, :], W3, b3[:, None, :])


# speedup vs baseline: 1.0588x; 1.0118x over previous
import functools

import jax
import jax.numpy as jnp
import numpy as np
from jax import lax
from jax.experimental import pallas as pl
from jax.experimental.pallas import tpu as pltpu
from jax.experimental.pallas import tpu_sc as plsc

_N_EXPERTS = 8
_N_ACTIVE = 5
_TEMP = float(np.e)
_N_TOK = 2048

_INFO = plsc.get_sparse_core_info()
_NC, _NS, _L = _INFO.num_cores, _INFO.num_subcores, _INFO.num_lanes
_NW = _NC * _NS
_TPW = _N_TOK // _NW


def _scores_body(x_ref, gw_ref, gb_ref, st_ref, xb_ref):
    xb_ref[...] = x_ref[...].astype(jnp.bfloat16)
    st_ref[...] = (lax.dot_general(
        gw_ref[...], x_ref[...], (((0,), (1,)), ((), ())),
        preferred_element_type=jnp.float32)
        + gb_ref[...].reshape(_N_EXPERTS, 1)) / _TEMP


_gate_mesh = plsc.VectorSubcoreMesh(core_axis_name="c", subcore_axis_name="s")


@functools.partial(
    pl.kernel, mesh=_gate_mesh,
    out_type=jax.ShapeDtypeStruct((_N_EXPERTS, _N_TOK), jnp.float32),
    scratch_types=[
        pltpu.VMEM((_N_EXPERTS, _TPW), jnp.float32),
        pltpu.VMEM((_N_EXPERTS, _TPW), jnp.float32),
        pltpu.SemaphoreType.DMA,
    ],
)
def _gate_sc(st_hbm, w_hbm, sc_v, w_v, sem):
    wid = lax.axis_index("s") * _NC + lax.axis_index("c")
    base = wid * _TPW
    copies = [pltpu.make_async_copy(
        st_hbm.at[i, pl.ds(base, _TPW)], sc_v.at[i], sem)
        for i in range(_N_EXPERTS)]
    for cp in copies:
        cp.start()
    for cp in copies:
        cp.wait()
    for c in range(_TPW // _L):
        sl = pl.ds(c * _L, _L)
        p = [sc_v[i, sl] for i in range(_N_EXPERTS)]
        m = p[0]
        for i in range(1, _N_EXPERTS):
            m = jnp.maximum(m, p[i])
        ex = [jnp.exp(v - m) for v in p]
        tot = ex[0]
        for i in range(1, _N_EXPERTS):
            tot = tot + ex[i]
        pr = [v / tot for v in ex]
        w_cols = []
        wsum = None
        for i in range(_N_EXPERTS):
            beats = None
            for j in range(_N_EXPERTS):
                if j == i:
                    continue
                if j < i:
                    b = jnp.where(pr[j] >= pr[i], 1.0, 0.0)
                else:
                    b = jnp.where(pr[j] > pr[i], 1.0, 0.0)
                beats = b if beats is None else beats + b
            wi = jnp.where(beats < float(_N_ACTIVE), pr[i], 0.0)
            w_cols.append(wi)
            wsum = wi if wsum is None else wsum + wi
        inv = 1.0 / (wsum + 1e-8)
        for i in range(_N_EXPERTS):
            w_v[i, sl] = w_cols[i] * inv
    out_copies = [pltpu.make_async_copy(
        w_v.at[i], w_hbm.at[i, pl.ds(base, _TPW)], sem)
        for i in range(_N_EXPERTS)]
    for cp in out_copies:
        cp.start()
    for cp in out_copies:
        cp.wait()


def _experts_body(xb_ref, wts_ref, w1_ref, b1_ref, w2_ref, b2_ref,
                  w3_ref, b3_ref, out_ref):
    e = pl.program_id(0)
    w1b = w1_ref[0].astype(jnp.bfloat16)
    w2b = w2_ref[0].astype(jnp.bfloat16)
    w3b = w3_ref[0].astype(jnp.bfloat16)

    h1 = jnp.maximum(
        jnp.dot(xb_ref[...], w1b, preferred_element_type=jnp.float32)
        + b1_ref[0], 0.0).astype(jnp.bfloat16)
    h2 = jnp.maximum(
        jnp.dot(h1, w2b, preferred_element_type=jnp.float32)
        + b2_ref[0], 0.0).astype(jnp.bfloat16)
    o = jnp.dot(h2, w3b, preferred_element_type=jnp.float32) + b3_ref[0]

    w_col = jnp.transpose(wts_ref[0])
    contrib = w_col * o

    @pl.when(e == 0)
    def _init():
        out_ref[...] = contrib

    @pl.when(e != 0)
    def _acc():
        out_ref[...] += contrib


def kernel(x, gate_W, gate_b, W1, b1, W2, b2, W3, b3):
    n, d = x.shape
    st, xb = pl.pallas_call(
        _scores_body,
        out_shape=[
            jax.ShapeDtypeStruct((_N_EXPERTS, n), jnp.float32),
            jax.ShapeDtypeStruct((n, d), jnp.bfloat16),
        ],
    )(x, gate_W, gate_b.reshape(1, -1))

    wts = _gate_sc(st)

    return pl.pallas_call(
        _experts_body,
        grid=(_N_EXPERTS,),
        in_specs=[
            pl.BlockSpec((_N_TOK, d), lambda e: (0, 0)),
            pl.BlockSpec((1, 1, _N_TOK), lambda e: (e, 0, 0)),
            pl.BlockSpec((1, d, W1.shape[2]), lambda e: (e, 0, 0)),
            pl.BlockSpec((1, 1, b1.shape[1]), lambda e: (e, 0, 0)),
            pl.BlockSpec((1, W2.shape[1], W2.shape[2]), lambda e: (e, 0, 0)),
            pl.BlockSpec((1, 1, b2.shape[1]), lambda e: (e, 0, 0)),
            pl.BlockSpec((1, W3.shape[1], W3.shape[2]), lambda e: (e, 0, 0)),
            pl.BlockSpec((1, 1, b3.shape[1]), lambda e: (e, 0, 0)),
        ],
        out_specs=pl.BlockSpec((_N_TOK, W3.shape[2]), lambda e: (0, 0)),
        out_shape=jax.ShapeDtypeStruct((n, W3.shape[2]), jnp.float32),
        compiler_params=pltpu.CompilerParams(
            dimension_semantics=("arbitrary",),
            vmem_limit_bytes=100 * 1024 * 1024,
        ),
    )(xb, wts[:, None, :], W1, b1[:, None, :], W2, b2[:, None, :], W3,
      b3[:, None, :])


# speedup vs baseline: 1.0761x; 1.0163x over previous
import functools

import jax
import jax.numpy as jnp
import numpy as np
from jax import lax
from jax.experimental import pallas as pl
from jax.experimental.pallas import tpu as pltpu
from jax.experimental.pallas import tpu_sc as plsc

_N_EXPERTS = 8
_N_ACTIVE = 5
_TEMP = float(np.e)
_N_TOK = 2048

_INFO = plsc.get_sparse_core_info()
_NC, _NS, _L = _INFO.num_cores, _INFO.num_subcores, _INFO.num_lanes
_NW = _NC * _NS
_TPW = _N_TOK // _NW


def _scores_body(x_ref, gw_ref, gb_ref, st_ref, xb_ref):
    xb_ref[...] = x_ref[...].astype(jnp.bfloat16)
    st_ref[...] = (lax.dot_general(
        gw_ref[...], x_ref[...], (((0,), (1,)), ((), ())),
        preferred_element_type=jnp.float32)
        + gb_ref[...].reshape(_N_EXPERTS, 1)) / _TEMP


_gate_mesh = plsc.VectorSubcoreMesh(core_axis_name="c", subcore_axis_name="s")


@functools.partial(
    pl.kernel, mesh=_gate_mesh,
    out_type=jax.ShapeDtypeStruct((_N_EXPERTS, _N_TOK), jnp.float32),
    scratch_types=[
        pltpu.VMEM((_N_EXPERTS, _TPW), jnp.float32),
        pltpu.VMEM((_N_EXPERTS, _TPW), jnp.float32),
        pltpu.SemaphoreType.DMA,
    ],
)
def _gate_sc(st_hbm, w_hbm, sc_v, w_v, sem):
    wid = lax.axis_index("s") * _NC + lax.axis_index("c")
    base = wid * _TPW
    copies = [pltpu.make_async_copy(
        st_hbm.at[i, pl.ds(base, _TPW)], sc_v.at[i], sem)
        for i in range(_N_EXPERTS)]
    for cp in copies:
        cp.start()
    for cp in copies:
        cp.wait()
    for c in range(_TPW // _L):
        sl = pl.ds(c * _L, _L)
        p = [sc_v[i, sl] for i in range(_N_EXPERTS)]
        m = p[0]
        for i in range(1, _N_EXPERTS):
            m = jnp.maximum(m, p[i])
        ex = [jnp.exp(v - m) for v in p]
        tot = ex[0]
        for i in range(1, _N_EXPERTS):
            tot = tot + ex[i]
        pr = [v / tot for v in ex]
        w_cols = []
        wsum = None
        for i in range(_N_EXPERTS):
            beats = None
            for j in range(_N_EXPERTS):
                if j == i:
                    continue
                if j < i:
                    b = jnp.where(pr[j] >= pr[i], 1.0, 0.0)
                else:
                    b = jnp.where(pr[j] > pr[i], 1.0, 0.0)
                beats = b if beats is None else beats + b
            wi = jnp.where(beats < float(_N_ACTIVE), pr[i], 0.0)
            w_cols.append(wi)
            wsum = wi if wsum is None else wsum + wi
        inv = 1.0 / (wsum + 1e-8)
        for i in range(_N_EXPERTS):
            w_v[i, sl] = w_cols[i] * inv
    out_copies = [pltpu.make_async_copy(
        w_v.at[i], w_hbm.at[i, pl.ds(base, _TPW)], sem)
        for i in range(_N_EXPERTS)]
    for cp in out_copies:
        cp.start()
    for cp in out_copies:
        cp.wait()


def _experts_body(xb_ref, wts_ref, b3f_ref, w1_ref, b1_ref, w2_ref,
                  b2_ref, w3_ref, b3_ref, out_ref):
    e = pl.program_id(0)

    @pl.when(e == 0)
    def _init():
        out_ref[...] = jnp.dot(jnp.transpose(wts_ref[...]), b3f_ref[...],
                               preferred_element_type=jnp.float32)

    w1b = w1_ref[0].astype(jnp.bfloat16)
    w2b = w2_ref[0].astype(jnp.bfloat16)
    w3b = w3_ref[0].astype(jnp.bfloat16)
    w_col = jnp.transpose(wts_ref[pl.ds(e, 1), :])

    h1 = jnp.maximum(
        jnp.dot(xb_ref[...], w1b, preferred_element_type=jnp.float32)
        + b1_ref[0], 0.0).astype(jnp.bfloat16)
    h2 = jnp.maximum(
        jnp.dot(h1, w2b, preferred_element_type=jnp.float32)
        + b2_ref[0], 0.0)
    h2w = (h2 * w_col).astype(jnp.bfloat16)
    out_ref[...] += jnp.dot(h2w, w3b, preferred_element_type=jnp.float32)


def kernel(x, gate_W, gate_b, W1, b1, W2, b2, W3, b3):
    n, d = x.shape
    st, xb = pl.pallas_call(
        _scores_body,
        out_shape=[
            jax.ShapeDtypeStruct((_N_EXPERTS, n), jnp.float32),
            jax.ShapeDtypeStruct((n, d), jnp.bfloat16),
        ],
    )(x, gate_W, gate_b.reshape(1, -1))

    wts = _gate_sc(st)

    return pl.pallas_call(
        _experts_body,
        grid=(_N_EXPERTS,),
        in_specs=[
            pl.BlockSpec((_N_TOK, d), lambda e: (0, 0)),
            pl.BlockSpec((_N_EXPERTS, _N_TOK), lambda e: (0, 0)),
            pl.BlockSpec((_N_EXPERTS, b3.shape[1]), lambda e: (0, 0)),
            pl.BlockSpec((1, d, W1.shape[2]), lambda e: (e, 0, 0)),
            pl.BlockSpec((1, 1, b1.shape[1]), lambda e: (e, 0, 0)),
            pl.BlockSpec((1, W2.shape[1], W2.shape[2]), lambda e: (e, 0, 0)),
            pl.BlockSpec((1, 1, b2.shape[1]), lambda e: (e, 0, 0)),
            pl.BlockSpec((1, W3.shape[1], W3.shape[2]), lambda e: (e, 0, 0)),
            pl.BlockSpec((1, 1, b3.shape[1]), lambda e: (e, 0, 0)),
        ],
        out_specs=pl.BlockSpec((_N_TOK, W3.shape[2]), lambda e: (0, 0)),
        out_shape=jax.ShapeDtypeStruct((n, W3.shape[2]), jnp.float32),
        compiler_params=pltpu.CompilerParams(
            dimension_semantics=("arbitrary",),
            vmem_limit_bytes=100 * 1024 * 1024,
        ),
    )(xb, wts, b3, W1, b1[:, None, :], W2, b2[:, None, :], W3,
      b3[:, None, :])


# speedup vs baseline: 1.0775x; 1.0012x over previous
import functools

import jax
import jax.numpy as jnp
import numpy as np
from jax import lax
from jax.experimental import pallas as pl
from jax.experimental.pallas import tpu as pltpu
from jax.experimental.pallas import tpu_sc as plsc

_N_EXPERTS = 8
_N_ACTIVE = 5
_TEMP = float(np.e)
_N_TOK = 2048

_INFO = plsc.get_sparse_core_info()
_NC, _NS, _L = _INFO.num_cores, _INFO.num_subcores, _INFO.num_lanes
_NW = _NC * _NS
_TPW = _N_TOK // _NW


def _scores_body(x_ref, gw_ref, gb_ref, st_ref):
    st_ref[...] = (lax.dot_general(
        gw_ref[...], x_ref[...], (((0,), (1,)), ((), ())),
        preferred_element_type=jnp.float32)
        + gb_ref[...].reshape(_N_EXPERTS, 1)) / _TEMP


_gate_mesh = plsc.VectorSubcoreMesh(core_axis_name="c", subcore_axis_name="s")


@functools.partial(
    pl.kernel, mesh=_gate_mesh,
    out_type=jax.ShapeDtypeStruct((_N_EXPERTS, _N_TOK), jnp.float32),
    scratch_types=[
        pltpu.VMEM((_N_EXPERTS, _TPW), jnp.float32),
        pltpu.VMEM((_N_EXPERTS, _TPW), jnp.float32),
        pltpu.SemaphoreType.DMA,
    ],
)
def _gate_sc(st_hbm, w_hbm, sc_v, w_v, sem):
    wid = lax.axis_index("s") * _NC + lax.axis_index("c")
    base = wid * _TPW
    copies = [pltpu.make_async_copy(
        st_hbm.at[i, pl.ds(base, _TPW)], sc_v.at[i], sem)
        for i in range(_N_EXPERTS)]
    for cp in copies:
        cp.start()
    for cp in copies:
        cp.wait()
    for c in range(_TPW // _L):
        sl = pl.ds(c * _L, _L)
        p = [sc_v[i, sl] for i in range(_N_EXPERTS)]
        m = p[0]
        for i in range(1, _N_EXPERTS):
            m = jnp.maximum(m, p[i])
        ex = [jnp.exp(v - m) for v in p]
        tot = ex[0]
        for i in range(1, _N_EXPERTS):
            tot = tot + ex[i]
        pr = [v / tot for v in ex]
        w_cols = []
        wsum = None
        for i in range(_N_EXPERTS):
            beats = None
            for j in range(_N_EXPERTS):
                if j == i:
                    continue
                if j < i:
                    b = jnp.where(pr[j] >= pr[i], 1.0, 0.0)
                else:
                    b = jnp.where(pr[j] > pr[i], 1.0, 0.0)
                beats = b if beats is None else beats + b
            wi = jnp.where(beats < float(_N_ACTIVE), pr[i], 0.0)
            w_cols.append(wi)
            wsum = wi if wsum is None else wsum + wi
        inv = 1.0 / (wsum + 1e-8)
        for i in range(_N_EXPERTS):
            w_v[i, sl] = w_cols[i] * inv
    out_copies = [pltpu.make_async_copy(
        w_v.at[i], w_hbm.at[i, pl.ds(base, _TPW)], sem)
        for i in range(_N_EXPERTS)]
    for cp in out_copies:
        cp.start()
    for cp in out_copies:
        cp.wait()


def _experts_body(xb_ref, wts_ref, b3f_ref, w1_ref, b1_ref, w2_ref,
                  b2_ref, w3_ref, b3_ref, out_ref):
    e = pl.program_id(0)

    @pl.when(e == 0)
    def _init():
        out_ref[...] = jnp.dot(jnp.transpose(wts_ref[...]), b3f_ref[...],
                               preferred_element_type=jnp.float32)

    w_col = jnp.transpose(wts_ref[pl.ds(e, 1), :])

    h1 = jnp.maximum(
        jnp.dot(xb_ref[...], w1_ref[0], preferred_element_type=jnp.float32)
        + b1_ref[0], 0.0)
    h2 = jnp.maximum(
        jnp.dot(h1, w2_ref[0], preferred_element_type=jnp.float32)
        + b2_ref[0], 0.0)
    h2w = h2 * w_col
    out_ref[...] += jnp.dot(h2w, w3_ref[0], preferred_element_type=jnp.float32)


def kernel(x, gate_W, gate_b, W1, b1, W2, b2, W3, b3):
    n, d = x.shape
    st = pl.pallas_call(
        _scores_body,
        out_shape=jax.ShapeDtypeStruct((_N_EXPERTS, n), jnp.float32),
    )(x, gate_W, gate_b.reshape(1, -1))

    wts = _gate_sc(st)

    return pl.pallas_call(
        _experts_body,
        grid=(_N_EXPERTS,),
        in_specs=[
            pl.BlockSpec((_N_TOK, d), lambda e: (0, 0)),
            pl.BlockSpec((_N_EXPERTS, _N_TOK), lambda e: (0, 0)),
            pl.BlockSpec((_N_EXPERTS, b3.shape[1]), lambda e: (0, 0)),
            pl.BlockSpec((1, d, W1.shape[2]), lambda e: (e, 0, 0)),
            pl.BlockSpec((1, 1, b1.shape[1]), lambda e: (e, 0, 0)),
            pl.BlockSpec((1, W2.shape[1], W2.shape[2]), lambda e: (e, 0, 0)),
            pl.BlockSpec((1, 1, b2.shape[1]), lambda e: (e, 0, 0)),
            pl.BlockSpec((1, W3.shape[1], W3.shape[2]), lambda e: (e, 0, 0)),
            pl.BlockSpec((1, 1, b3.shape[1]), lambda e: (e, 0, 0)),
        ],
        out_specs=pl.BlockSpec((_N_TOK, W3.shape[2]), lambda e: (0, 0)),
        out_shape=jax.ShapeDtypeStruct((n, W3.shape[2]), jnp.float32),
        compiler_params=pltpu.CompilerParams(
            dimension_semantics=("arbitrary",),
            vmem_limit_bytes=100 * 1024 * 1024,
        ),
    )(x, wts, b3, W1, b1[:, None, :], W2, b2[:, None, :], W3,
      b3[:, None, :])
